# Initial kernel scaffold; baseline (speedup 1.0000x reference)
import jax
import jax.numpy as jnp
from jax.experimental import pallas as pl


def kernel(input, adj, W):
    raise NotImplementedError("write your pallas kernel here")


# speedup vs baseline: 1.9778x; 1.9778x over previous
import jax
import jax.numpy as jnp
from jax.experimental import pallas as pl

_BM = 400


def _fused_body(xblk_ref, adj_ref, x_ref, w_ref, o_ref):
    adj = adj_ref[...]
    h = jnp.dot(adj, x_ref[...], preferred_element_type=jnp.float32)
    deg = jnp.sum(adj, axis=1, keepdims=True) + 1e-6
    f = x_ref.shape[1]
    w_top = w_ref[:f, :]
    w_bot = w_ref[f:, :]
    self_term = jnp.dot(xblk_ref[...], w_top, preferred_element_type=jnp.float32)
    agg_term = jnp.dot(h / deg, w_bot, preferred_element_type=jnp.float32)
    o_ref[...] = self_term + agg_term


def kernel(input, adj, W):
    n, f = input.shape
    out_f = W.shape[1]
    grid = (n // _BM,)
    return pl.pallas_call(
        _fused_body,
        grid=grid,
        in_specs=[
            pl.BlockSpec((_BM, f), lambda i: (i, 0)),
            pl.BlockSpec((_BM, n), lambda i: (i, 0)),
            pl.BlockSpec((n, f), lambda i: (0, 0)),
            pl.BlockSpec(W.shape, lambda i: (0, 0)),
        ],
        out_specs=pl.BlockSpec((_BM, out_f), lambda i: (i, 0)),
        out_shape=jax.ShapeDtypeStruct((n, out_f), jnp.float32),
    )(input, adj, input, W)
